# Initial kernel scaffold; baseline (speedup 1.0000x reference)
import functools

import jax
import jax.numpy as jnp
from jax import lax
from jax.experimental import pallas as pl
from jax.experimental.pallas import tpu as pltpu
from jax.experimental.pallas import tpu_sc as plsc

N_NODES = 10000
N_EDGES = 160000
N_TILES = 16
ROWS_PER_TILE = N_NODES // N_TILES
EDGE_ROWS = 1280
EDGE_PAD = EDGE_ROWS * 128
ROWS_PER_TILE_E = EDGE_ROWS // N_TILES
BLK_IDX_ROWS = 4
N_OUTER = ROWS_PER_TILE_E // BLK_IDX_ROWS


def _sc_aggregate(xflat, src_off, dst2, nch):
    mesh = plsc.VectorSubcoreMesh(core_axis_name="c", subcore_axis_name="s")
    out_t = jax.ShapeDtypeStruct((nch * N_NODES, 128), jnp.float32)
    scratch = [
        pltpu.VMEM((BLK_IDX_ROWS, 128), jnp.int32),
        pltpu.VMEM((BLK_IDX_ROWS, 128), jnp.int32),
        pltpu.VMEM((BLK_IDX_ROWS * 128, 128), jnp.float32),
        pltpu.VMEM_SHARED((N_NODES + 8, 128), jnp.float32),
    ]

    @functools.partial(pl.kernel, out_type=out_t, mesh=mesh,
                       scratch_types=scratch)
    def k(x_hbm, s_hbm, d_hbm, o_hbm, idx_s, idx_d, rows, accum):
        core = lax.axis_index("c")
        tid = lax.axis_index("s")
        for chunk in range(nch):
            @pl.when(core == (chunk % 2))
            def _():
                r0 = tid * ROWS_PER_TILE
                pltpu.sync_copy(
                    x_hbm.at[pl.ds(chunk * N_NODES + r0, ROWS_PER_TILE), :],
                    accum.at[pl.ds(r0, ROWS_PER_TILE), :])
                plsc.subcore_barrier()

                @pl.loop(0, N_OUTER)
                def _(it):
                    base = tid * ROWS_PER_TILE_E + it * BLK_IDX_ROWS
                    pltpu.sync_copy(
                        s_hbm.at[chunk, pl.ds(base, BLK_IDX_ROWS), :], idx_s)
                    pltpu.sync_copy(
                        d_hbm.at[pl.ds(base, BLK_IDX_ROWS), :], idx_d)
                    for j in range(BLK_IDX_ROWS):
                        pltpu.sync_copy(x_hbm.at[idx_s.at[j]],
                                        rows.at[pl.ds(j * 128, 128), :])
                    for j in range(BLK_IDX_ROWS):
                        pltpu.sync_copy(rows.at[pl.ds(j * 128, 128), :],
                                        accum.at[idx_d.at[j]], add=True)

                plsc.subcore_barrier()
                pltpu.sync_copy(
                    accum.at[pl.ds(r0, ROWS_PER_TILE), :],
                    o_hbm.at[pl.ds(chunk * N_NODES + r0, ROWS_PER_TILE), :])

    return k(xflat, src_off, dst2)


def _tc_mlp(hpre, W1, b1, W2, b2):
    nchin = hpre.shape[0]
    h_mid = W1.shape[1]
    dout = W2.shape[1]
    nchout = dout // 128
    R = 1000
    G = N_NODES // R

    def body(h_ref, w1_ref, b1_ref, w2_ref, b2_ref, o_ref):
        s = jnp.dot(h_ref[0], w1_ref[0:128, :],
                    preferred_element_type=jnp.float32)
        for c in range(1, nchin):
            s = s + jnp.dot(h_ref[c], w1_ref[128 * c:128 * (c + 1), :],
                            preferred_element_type=jnp.float32)
        m = jnp.maximum(s + b1_ref[...], 0.0)
        o = jnp.dot(m, w2_ref[...], preferred_element_type=jnp.float32)
        o = o + b2_ref[...]
        for c in range(nchout):
            o_ref[c, :, :] = o[:, 128 * c:128 * (c + 1)]

    return pl.pallas_call(
        body,
        grid=(G,),
        in_specs=[
            pl.BlockSpec((nchin, R, 128), lambda i: (0, i, 0)),
            pl.BlockSpec((nchin * 128, h_mid), lambda i: (0, 0)),
            pl.BlockSpec((1, h_mid), lambda i: (0, 0)),
            pl.BlockSpec((h_mid, dout), lambda i: (0, 0)),
            pl.BlockSpec((1, dout), lambda i: (0, 0)),
        ],
        out_specs=pl.BlockSpec((nchout, R, 128), lambda i: (0, i, 0)),
        out_shape=jax.ShapeDtypeStruct((nchout, N_NODES, 128), jnp.float32),
    )(hpre, W1, b1.reshape(1, h_mid), W2, b2.reshape(1, dout))


def _tc_final(hpre, W3a, b3a, W3b, b3b):
    nchin = hpre.shape[0]
    h_mid = W3a.shape[1]
    dout = W3b.shape[1]
    R = 1000
    G = N_NODES // R

    def body(h_ref, w3a_ref, b3a_ref, w3b_ref, b3b_ref, o_ref, acc_ref):
        i = pl.program_id(0)
        s = jnp.dot(h_ref[0], w3a_ref[0:128, :],
                    preferred_element_type=jnp.float32)
        for c in range(1, nchin):
            s = s + jnp.dot(h_ref[c], w3a_ref[128 * c:128 * (c + 1), :],
                            preferred_element_type=jnp.float32)
        m = jnp.maximum(s + b3a_ref[...], 0.0)
        part = jnp.sum(m, axis=0, keepdims=True)

        @pl.when(i == 0)
        def _():
            acc_ref[...] = part

        @pl.when(i > 0)
        def _():
            acc_ref[...] = acc_ref[...] + part

        @pl.when(i == G - 1)
        def _():
            o_ref[...] = (jnp.dot(acc_ref[...], w3b_ref[...],
                                  preferred_element_type=jnp.float32)
                          + float(N_NODES) * b3b_ref[...])

    return pl.pallas_call(
        body,
        grid=(G,),
        in_specs=[
            pl.BlockSpec((nchin, R, 128), lambda i: (0, i, 0)),
            pl.BlockSpec((nchin * 128, h_mid), lambda i: (0, 0)),
            pl.BlockSpec((1, h_mid), lambda i: (0, 0)),
            pl.BlockSpec((h_mid, dout), lambda i: (0, 0)),
            pl.BlockSpec((1, dout), lambda i: (0, 0)),
        ],
        out_specs=pl.BlockSpec((1, dout), lambda i: (0, 0)),
        out_shape=jax.ShapeDtypeStruct((1, dout), jnp.float32),
        scratch_shapes=[pltpu.VMEM((1, h_mid), jnp.float32)],
    )(hpre, W3a, b3a.reshape(1, h_mid), W3b, b3b.reshape(1, dout))


def kernel(x, edge_index, edge_attr, W1a, b1a, W1b, b1b, W2a, b2a, W2b, b2b,
           W3a, b3a, W3b, b3b):
    del edge_attr
    src = edge_index[0].astype(jnp.int32)
    dst = edge_index[1].astype(jnp.int32)
    pad = EDGE_PAD - N_EDGES
    srcp = jnp.concatenate([src, jnp.zeros((pad,), jnp.int32)])
    dstp = jnp.concatenate([dst, jnp.full((pad,), N_NODES, jnp.int32)])
    dst2 = dstp.reshape(EDGE_ROWS, 128)
    src2 = srcp.reshape(EDGE_ROWS, 128)
    off2 = (jnp.arange(2, dtype=jnp.int32) * N_NODES)[:, None, None]
    off4 = (jnp.arange(4, dtype=jnp.int32) * N_NODES)[:, None, None]
    src_off2 = src2[None] + off2
    src_off4 = src2[None] + off4

    x2 = x.reshape(N_NODES, 2, 128).transpose(1, 0, 2)

    hpre1 = _sc_aggregate(x2.reshape(2 * N_NODES, 128), src_off2, dst2, 2)
    h1 = _tc_mlp(hpre1.reshape(2, N_NODES, 128), W1a, b1a, W1b, b1b)

    hpre2 = _sc_aggregate(h1.reshape(4 * N_NODES, 128), src_off4, dst2, 4)
    h2 = _tc_mlp(hpre2.reshape(4, N_NODES, 128), W2a, b2a, W2b, b2b)

    hpre3 = _sc_aggregate(h2.reshape(4 * N_NODES, 128), src_off4, dst2, 4)
    out = _tc_final(hpre3.reshape(4, N_NODES, 128), W3a, b3a, W3b, b3b)
    return out


# speedup vs baseline: 2.6003x; 2.6003x over previous
import functools

import jax
import jax.numpy as jnp
from jax import lax
from jax.experimental import pallas as pl
from jax.experimental.pallas import tpu as pltpu
from jax.experimental.pallas import tpu_sc as plsc

N_NODES = 10000
N_EDGES = 160000
N_TILES = 16
ROWS_PER_TILE = 624
TAIL_ROWS = N_NODES - N_TILES * ROWS_PER_TILE
EDGE_ROWS = 1280
EDGE_PAD = EDGE_ROWS * 128
ROWS_PER_TILE_E = EDGE_ROWS // N_TILES
BLK_IDX_ROWS = 8
N_OUTER = ROWS_PER_TILE_E // BLK_IDX_ROWS


def _sc_aggregate(xflat, src_off, dst2, nch):
    mesh = plsc.VectorSubcoreMesh(core_axis_name="c", subcore_axis_name="s")
    out_t = jax.ShapeDtypeStruct((nch * N_NODES, 128), jnp.float32)
    scratch = [
        pltpu.VMEM((BLK_IDX_ROWS, 128), jnp.int32),
        pltpu.VMEM((BLK_IDX_ROWS, 128), jnp.int32),
        pltpu.VMEM((256, 128), jnp.float32),
        pltpu.VMEM_SHARED((N_NODES + 8, 128), jnp.float32),
    ]

    @functools.partial(pl.kernel, out_type=out_t, mesh=mesh,
                       scratch_types=scratch)
    def k(x_hbm, s_hbm, d_hbm, o_hbm, idx_s, idx_d, rows, accum):
        core = lax.axis_index("c")
        tid = lax.axis_index("s")
        for chunk in range(nch):
            @pl.when(core == (chunk % 2))
            def _():
                r0 = tid * ROWS_PER_TILE
                pltpu.sync_copy(
                    x_hbm.at[pl.ds(chunk * N_NODES + r0, ROWS_PER_TILE), :],
                    accum.at[pl.ds(r0, ROWS_PER_TILE), :])

                @pl.when(tid == 0)
                def _():
                    t0 = N_TILES * ROWS_PER_TILE
                    pltpu.sync_copy(
                        x_hbm.at[pl.ds(chunk * N_NODES + t0, TAIL_ROWS), :],
                        accum.at[pl.ds(t0, TAIL_ROWS), :])

                plsc.subcore_barrier()

                @pl.loop(0, N_OUTER)
                def _(it):
                    base = tid * ROWS_PER_TILE_E + it * BLK_IDX_ROWS
                    pltpu.sync_copy(
                        s_hbm.at[chunk, pl.ds(base, BLK_IDX_ROWS), :], idx_s)
                    pltpu.sync_copy(
                        d_hbm.at[pl.ds(base, BLK_IDX_ROWS), :], idx_d)
                    for quarter in range(4):
                        for j in range(2):
                            jj = quarter * 2 + j
                            pltpu.sync_copy(x_hbm.at[idx_s.at[jj]],
                                            rows.at[pl.ds(j * 128, 128), :])
                        for j in range(2):
                            jj = quarter * 2 + j
                            pltpu.sync_copy(rows.at[pl.ds(j * 128, 128), :],
                                            accum.at[idx_d.at[jj]], add=True)

                plsc.subcore_barrier()
                pltpu.sync_copy(
                    accum.at[pl.ds(r0, ROWS_PER_TILE), :],
                    o_hbm.at[pl.ds(chunk * N_NODES + r0, ROWS_PER_TILE), :])

                @pl.when(tid == 0)
                def _():
                    t0 = N_TILES * ROWS_PER_TILE
                    pltpu.sync_copy(
                        accum.at[pl.ds(t0, TAIL_ROWS), :],
                        o_hbm.at[pl.ds(chunk * N_NODES + t0, TAIL_ROWS), :])

    return k(xflat, src_off, dst2)


def _tc_mlp(hpre, W1, b1, W2, b2):
    nchin = hpre.shape[0]
    h_mid = W1.shape[1]
    dout = W2.shape[1]
    nchout = dout // 128
    R = 1000
    G = N_NODES // R

    def body(h_ref, w1_ref, b1_ref, w2_ref, b2_ref, o_ref):
        s = jnp.dot(h_ref[0], w1_ref[0:128, :],
                    preferred_element_type=jnp.float32)
        for c in range(1, nchin):
            s = s + jnp.dot(h_ref[c], w1_ref[128 * c:128 * (c + 1), :],
                            preferred_element_type=jnp.float32)
        m = jnp.maximum(s + b1_ref[...], 0.0)
        o = jnp.dot(m, w2_ref[...], preferred_element_type=jnp.float32)
        o = o + b2_ref[...]
        for c in range(nchout):
            o_ref[c, :, :] = o[:, 128 * c:128 * (c + 1)]

    return pl.pallas_call(
        body,
        grid=(G,),
        in_specs=[
            pl.BlockSpec((nchin, R, 128), lambda i: (0, i, 0)),
            pl.BlockSpec((nchin * 128, h_mid), lambda i: (0, 0)),
            pl.BlockSpec((1, h_mid), lambda i: (0, 0)),
            pl.BlockSpec((h_mid, dout), lambda i: (0, 0)),
            pl.BlockSpec((1, dout), lambda i: (0, 0)),
        ],
        out_specs=pl.BlockSpec((nchout, R, 128), lambda i: (0, i, 0)),
        out_shape=jax.ShapeDtypeStruct((nchout, N_NODES, 128), jnp.float32),
    )(hpre, W1, b1.reshape(1, h_mid), W2, b2.reshape(1, dout))


def _tc_final(hpre, W3a, b3a, W3b, b3b):
    nchin = hpre.shape[0]
    h_mid = W3a.shape[1]
    dout = W3b.shape[1]
    R = 1000
    G = N_NODES // R

    def body(h_ref, w3a_ref, b3a_ref, w3b_ref, b3b_ref, o_ref, acc_ref):
        i = pl.program_id(0)
        s = jnp.dot(h_ref[0], w3a_ref[0:128, :],
                    preferred_element_type=jnp.float32)
        for c in range(1, nchin):
            s = s + jnp.dot(h_ref[c], w3a_ref[128 * c:128 * (c + 1), :],
                            preferred_element_type=jnp.float32)
        m = jnp.maximum(s + b3a_ref[...], 0.0)
        part = jnp.sum(m, axis=0, keepdims=True)

        @pl.when(i == 0)
        def _():
            acc_ref[...] = part

        @pl.when(i > 0)
        def _():
            acc_ref[...] = acc_ref[...] + part

        @pl.when(i == G - 1)
        def _():
            o_ref[...] = (jnp.dot(acc_ref[...], w3b_ref[...],
                                  preferred_element_type=jnp.float32)
                          + float(N_NODES) * b3b_ref[...])

    return pl.pallas_call(
        body,
        grid=(G,),
        in_specs=[
            pl.BlockSpec((nchin, R, 128), lambda i: (0, i, 0)),
            pl.BlockSpec((nchin * 128, h_mid), lambda i: (0, 0)),
            pl.BlockSpec((1, h_mid), lambda i: (0, 0)),
            pl.BlockSpec((h_mid, dout), lambda i: (0, 0)),
            pl.BlockSpec((1, dout), lambda i: (0, 0)),
        ],
        out_specs=pl.BlockSpec((1, dout), lambda i: (0, 0)),
        out_shape=jax.ShapeDtypeStruct((1, dout), jnp.float32),
        scratch_shapes=[pltpu.VMEM((1, h_mid), jnp.float32)],
    )(hpre, W3a, b3a.reshape(1, h_mid), W3b, b3b.reshape(1, dout))


def kernel(x, edge_index, edge_attr, W1a, b1a, W1b, b1b, W2a, b2a, W2b, b2b,
           W3a, b3a, W3b, b3b):
    del edge_attr
    src = edge_index[0].astype(jnp.int32)
    dst = edge_index[1].astype(jnp.int32)
    pad = EDGE_PAD - N_EDGES
    srcp = jnp.concatenate([src, jnp.zeros((pad,), jnp.int32)])
    dstp = jnp.concatenate([dst, jnp.full((pad,), N_NODES, jnp.int32)])
    dst2 = dstp.reshape(EDGE_ROWS, 128)
    src2 = srcp.reshape(EDGE_ROWS, 128)
    off2 = (jnp.arange(2, dtype=jnp.int32) * N_NODES)[:, None, None]
    off4 = (jnp.arange(4, dtype=jnp.int32) * N_NODES)[:, None, None]
    src_off2 = src2[None] + off2
    src_off4 = src2[None] + off4

    x2 = x.reshape(N_NODES, 2, 128).transpose(1, 0, 2)

    hpre1 = _sc_aggregate(x2.reshape(2 * N_NODES, 128), src_off2, dst2, 2)
    h1 = _tc_mlp(hpre1.reshape(2, N_NODES, 128), W1a, b1a, W1b, b1b)

    hpre2 = _sc_aggregate(h1.reshape(4 * N_NODES, 128), src_off4, dst2, 4)
    h2 = _tc_mlp(hpre2.reshape(4, N_NODES, 128), W2a, b2a, W2b, b2b)

    hpre3 = _sc_aggregate(h2.reshape(4 * N_NODES, 128), src_off4, dst2, 4)
    out = _tc_final(hpre3.reshape(4, N_NODES, 128), W3a, b3a, W3b, b3b)
    return out


# speedup vs baseline: 3.2270x; 1.2410x over previous
import functools

import jax
import jax.numpy as jnp
from jax import lax
from jax.experimental import pallas as pl
from jax.experimental.pallas import tpu as pltpu
from jax.experimental.pallas import tpu_sc as plsc

N_NODES = 10000
N_EDGES = 160000
N_TILES = 16
ROWS_PER_TILE = 624
TAIL_ROWS = N_NODES - N_TILES * ROWS_PER_TILE
EDGE_ROWS = 1280
EDGE_PAD = EDGE_ROWS * 128
ROWS_PER_TILE_E = EDGE_ROWS // N_TILES
HALF_ROWS = ROWS_PER_TILE_E // 2


def _sc_aggregate(xflat, src_off, dst2, nch):
    mesh = plsc.VectorSubcoreMesh(core_axis_name="c", subcore_axis_name="s")
    out_t = jax.ShapeDtypeStruct((nch * N_NODES, 128), jnp.float32)
    scratch = [
        pltpu.VMEM((HALF_ROWS, 128), jnp.int32),
        pltpu.VMEM((HALF_ROWS, 128), jnp.int32),
        pltpu.VMEM((2, 128, 128), jnp.float32),
        pltpu.VMEM_SHARED((N_NODES + 8, 128), jnp.float32),
        pltpu.SemaphoreType.DMA,
        pltpu.SemaphoreType.DMA,
        pltpu.SemaphoreType.DMA,
        pltpu.SemaphoreType.DMA,
    ]

    @functools.partial(pl.kernel, out_type=out_t, mesh=mesh,
                       scratch_types=scratch)
    def k(x_hbm, s_hbm, d_hbm, o_hbm, idx_s, idx_d, rows, accum,
          sem_g0, sem_g1, sem_s0, sem_s1):
        core = lax.axis_index("c")
        tid = lax.axis_index("s")
        for chunk in range(nch):
            @pl.when(core == (chunk % 2))
            def _():
                r0 = tid * ROWS_PER_TILE
                pltpu.sync_copy(
                    x_hbm.at[pl.ds(chunk * N_NODES + r0, ROWS_PER_TILE), :],
                    accum.at[pl.ds(r0, ROWS_PER_TILE), :])

                @pl.when(tid == 0)
                def _():
                    t0 = N_TILES * ROWS_PER_TILE
                    pltpu.sync_copy(
                        x_hbm.at[pl.ds(chunk * N_NODES + t0, TAIL_ROWS), :],
                        accum.at[pl.ds(t0, TAIL_ROWS), :])

                plsc.subcore_barrier()

                for half in range(2):
                    hb = tid * ROWS_PER_TILE_E + half * HALF_ROWS
                    pltpu.sync_copy(
                        s_hbm.at[chunk, pl.ds(hb, HALF_ROWS), :], idx_s)
                    pltpu.sync_copy(
                        d_hbm.at[pl.ds(hb, HALF_ROWS), :], idx_d)
                    pltpu.async_copy(x_hbm.at[idx_s.at[0]], rows.at[0],
                                     sem_g0)
                    pltpu.async_copy(x_hbm.at[idx_s.at[1]], rows.at[1],
                                     sem_g1)

                    @pl.loop(0, HALF_ROWS // 2)
                    def _(rr):
                        r0 = 2 * rr
                        for b, sg, ss in ((0, sem_g0, sem_s0),
                                          (1, sem_g1, sem_s1)):
                            r = r0 + b
                            pltpu.make_async_copy(
                                x_hbm.at[pl.ds(0, 128), :], rows.at[b],
                                sg).wait()
                            pltpu.async_copy(rows.at[b],
                                             accum.at[idx_d.at[r]], ss,
                                             add=True)

                            @pl.when(rr < HALF_ROWS // 2 - 1)
                            def _():
                                pltpu.make_async_copy(
                                    x_hbm.at[pl.ds(0, 128), :], rows.at[b],
                                    ss).wait()
                                pltpu.async_copy(x_hbm.at[idx_s.at[r + 2]],
                                                 rows.at[b], sg)

                    pltpu.make_async_copy(x_hbm.at[pl.ds(0, 128), :],
                                          rows.at[0], sem_s0).wait()
                    pltpu.make_async_copy(x_hbm.at[pl.ds(0, 128), :],
                                          rows.at[1], sem_s1).wait()

                plsc.subcore_barrier()
                pltpu.sync_copy(
                    accum.at[pl.ds(r0, ROWS_PER_TILE), :],
                    o_hbm.at[pl.ds(chunk * N_NODES + r0, ROWS_PER_TILE), :])

                @pl.when(tid == 0)
                def _():
                    t0 = N_TILES * ROWS_PER_TILE
                    pltpu.sync_copy(
                        accum.at[pl.ds(t0, TAIL_ROWS), :],
                        o_hbm.at[pl.ds(chunk * N_NODES + t0, TAIL_ROWS), :])

    return k(xflat, src_off, dst2)


def _tc_mlp(hpre, W1, b1, W2, b2):
    nchin = hpre.shape[0]
    h_mid = W1.shape[1]
    dout = W2.shape[1]
    nchout = dout // 128
    R = 1000
    G = N_NODES // R

    def body(h_ref, w1_ref, b1_ref, w2_ref, b2_ref, o_ref):
        s = jnp.dot(h_ref[0], w1_ref[0:128, :],
                    preferred_element_type=jnp.float32)
        for c in range(1, nchin):
            s = s + jnp.dot(h_ref[c], w1_ref[128 * c:128 * (c + 1), :],
                            preferred_element_type=jnp.float32)
        m = jnp.maximum(s + b1_ref[...], 0.0)
        o = jnp.dot(m, w2_ref[...], preferred_element_type=jnp.float32)
        o = o + b2_ref[...]
        for c in range(nchout):
            o_ref[c, :, :] = o[:, 128 * c:128 * (c + 1)]

    return pl.pallas_call(
        body,
        grid=(G,),
        in_specs=[
            pl.BlockSpec((nchin, R, 128), lambda i: (0, i, 0)),
            pl.BlockSpec((nchin * 128, h_mid), lambda i: (0, 0)),
            pl.BlockSpec((1, h_mid), lambda i: (0, 0)),
            pl.BlockSpec((h_mid, dout), lambda i: (0, 0)),
            pl.BlockSpec((1, dout), lambda i: (0, 0)),
        ],
        out_specs=pl.BlockSpec((nchout, R, 128), lambda i: (0, i, 0)),
        out_shape=jax.ShapeDtypeStruct((nchout, N_NODES, 128), jnp.float32),
    )(hpre, W1, b1.reshape(1, h_mid), W2, b2.reshape(1, dout))


def _tc_final(hpre, W3a, b3a, W3b, b3b):
    nchin = hpre.shape[0]
    h_mid = W3a.shape[1]
    dout = W3b.shape[1]
    R = 1000
    G = N_NODES // R

    def body(h_ref, w3a_ref, b3a_ref, w3b_ref, b3b_ref, o_ref, acc_ref):
        i = pl.program_id(0)
        s = jnp.dot(h_ref[0], w3a_ref[0:128, :],
                    preferred_element_type=jnp.float32)
        for c in range(1, nchin):
            s = s + jnp.dot(h_ref[c], w3a_ref[128 * c:128 * (c + 1), :],
                            preferred_element_type=jnp.float32)
        m = jnp.maximum(s + b3a_ref[...], 0.0)
        part = jnp.sum(m, axis=0, keepdims=True)

        @pl.when(i == 0)
        def _():
            acc_ref[...] = part

        @pl.when(i > 0)
        def _():
            acc_ref[...] = acc_ref[...] + part

        @pl.when(i == G - 1)
        def _():
            o_ref[...] = (jnp.dot(acc_ref[...], w3b_ref[...],
                                  preferred_element_type=jnp.float32)
                          + float(N_NODES) * b3b_ref[...])

    return pl.pallas_call(
        body,
        grid=(G,),
        in_specs=[
            pl.BlockSpec((nchin, R, 128), lambda i: (0, i, 0)),
            pl.BlockSpec((nchin * 128, h_mid), lambda i: (0, 0)),
            pl.BlockSpec((1, h_mid), lambda i: (0, 0)),
            pl.BlockSpec((h_mid, dout), lambda i: (0, 0)),
            pl.BlockSpec((1, dout), lambda i: (0, 0)),
        ],
        out_specs=pl.BlockSpec((1, dout), lambda i: (0, 0)),
        out_shape=jax.ShapeDtypeStruct((1, dout), jnp.float32),
        scratch_shapes=[pltpu.VMEM((1, h_mid), jnp.float32)],
    )(hpre, W3a, b3a.reshape(1, h_mid), W3b, b3b.reshape(1, dout))


def kernel(x, edge_index, edge_attr, W1a, b1a, W1b, b1b, W2a, b2a, W2b, b2b,
           W3a, b3a, W3b, b3b):
    del edge_attr
    src = edge_index[0].astype(jnp.int32)
    dst = edge_index[1].astype(jnp.int32)
    pad = EDGE_PAD - N_EDGES
    srcp = jnp.concatenate([src, jnp.zeros((pad,), jnp.int32)])
    dstp = jnp.concatenate([dst, jnp.full((pad,), N_NODES, jnp.int32)])
    dst2 = dstp.reshape(EDGE_ROWS, 128)
    src2 = srcp.reshape(EDGE_ROWS, 128)
    off2 = (jnp.arange(2, dtype=jnp.int32) * N_NODES)[:, None, None]
    off4 = (jnp.arange(4, dtype=jnp.int32) * N_NODES)[:, None, None]
    src_off2 = src2[None] + off2
    src_off4 = src2[None] + off4

    x2 = x.reshape(N_NODES, 2, 128).transpose(1, 0, 2)

    hpre1 = _sc_aggregate(x2.reshape(2 * N_NODES, 128), src_off2, dst2, 2)
    h1 = _tc_mlp(hpre1.reshape(2, N_NODES, 128), W1a, b1a, W1b, b1b)

    hpre2 = _sc_aggregate(h1.reshape(4 * N_NODES, 128), src_off4, dst2, 4)
    h2 = _tc_mlp(hpre2.reshape(4, N_NODES, 128), W2a, b2a, W2b, b2b)

    hpre3 = _sc_aggregate(h2.reshape(4 * N_NODES, 128), src_off4, dst2, 4)
    out = _tc_final(hpre3.reshape(4, N_NODES, 128), W3a, b3a, W3b, b3b)
    return out


# speedup vs baseline: 3.2375x; 1.0033x over previous
import functools

import jax
import jax.numpy as jnp
from jax import lax
from jax.experimental import pallas as pl
from jax.experimental.pallas import tpu as pltpu
from jax.experimental.pallas import tpu_sc as plsc

N_NODES = 10000
N_EDGES = 160000
N_TILES = 16
ROWS_PER_TILE = 624
TAIL_ROWS = N_NODES - N_TILES * ROWS_PER_TILE
EDGE_ROWS = 1280
EDGE_PAD = EDGE_ROWS * 128
ROWS_PER_TILE_E = EDGE_ROWS // N_TILES
HALF_ROWS = ROWS_PER_TILE_E // 2


def _sc_aggregate(xflat, src_off, dst2, nch):
    mesh = plsc.VectorSubcoreMesh(core_axis_name="c", subcore_axis_name="s")
    out_t = jax.ShapeDtypeStruct((nch * N_NODES, 128), jnp.float32)
    scratch = [
        pltpu.VMEM((HALF_ROWS, 128), jnp.int32),
        pltpu.VMEM((HALF_ROWS, 128), jnp.int32),
        pltpu.VMEM((2, 128, 128), jnp.float32),
        pltpu.VMEM_SHARED((N_NODES + 8, 128), jnp.float32),
        pltpu.SemaphoreType.DMA,
        pltpu.SemaphoreType.DMA,
        pltpu.SemaphoreType.DMA,
        pltpu.SemaphoreType.DMA,
    ]

    @functools.partial(pl.kernel, out_type=out_t, mesh=mesh,
                       scratch_types=scratch)
    def k(x_hbm, s_hbm, d_hbm, o_hbm, idx_s, idx_d, rows, accum,
          sem_g0, sem_g1, sem_s0, sem_s1):
        core = lax.axis_index("c")
        tid = lax.axis_index("s")
        for chunk in range(nch):
            @pl.when(core == (chunk % 2))
            def _():
                r0 = tid * ROWS_PER_TILE
                pltpu.sync_copy(
                    x_hbm.at[pl.ds(chunk * N_NODES + r0, ROWS_PER_TILE), :],
                    accum.at[pl.ds(r0, ROWS_PER_TILE), :])

                @pl.when(tid == 0)
                def _():
                    t0 = N_TILES * ROWS_PER_TILE
                    pltpu.sync_copy(
                        x_hbm.at[pl.ds(chunk * N_NODES + t0, TAIL_ROWS), :],
                        accum.at[pl.ds(t0, TAIL_ROWS), :])

                plsc.subcore_barrier()

                for half in range(2):
                    hb = tid * ROWS_PER_TILE_E + half * HALF_ROWS
                    pltpu.sync_copy(
                        s_hbm.at[chunk, pl.ds(hb, HALF_ROWS), :], idx_s)
                    pltpu.sync_copy(
                        d_hbm.at[pl.ds(hb, HALF_ROWS), :], idx_d)
                    pltpu.async_copy(x_hbm.at[idx_s.at[0]], rows.at[0],
                                     sem_g0)
                    pltpu.async_copy(x_hbm.at[idx_s.at[1]], rows.at[1],
                                     sem_g1)

                    @pl.loop(0, HALF_ROWS // 2)
                    def _(rr):
                        r0 = 2 * rr
                        for b, sg, ss in ((0, sem_g0, sem_s0),
                                          (1, sem_g1, sem_s1)):
                            r = r0 + b
                            pltpu.make_async_copy(
                                x_hbm.at[pl.ds(0, 128), :], rows.at[b],
                                sg).wait()
                            pltpu.async_copy(rows.at[b],
                                             accum.at[idx_d.at[r]], ss,
                                             add=True)

                            @pl.when(rr < HALF_ROWS // 2 - 1)
                            def _():
                                pltpu.make_async_copy(
                                    x_hbm.at[pl.ds(0, 128), :], rows.at[b],
                                    ss).wait()
                                pltpu.async_copy(x_hbm.at[idx_s.at[r + 2]],
                                                 rows.at[b], sg)

                    pltpu.make_async_copy(x_hbm.at[pl.ds(0, 128), :],
                                          rows.at[0], sem_s0).wait()
                    pltpu.make_async_copy(x_hbm.at[pl.ds(0, 128), :],
                                          rows.at[1], sem_s1).wait()

                plsc.subcore_barrier()
                pltpu.sync_copy(
                    accum.at[pl.ds(r0, ROWS_PER_TILE), :],
                    o_hbm.at[pl.ds(chunk * N_NODES + r0, ROWS_PER_TILE), :])

                @pl.when(tid == 0)
                def _():
                    t0 = N_TILES * ROWS_PER_TILE
                    pltpu.sync_copy(
                        accum.at[pl.ds(t0, TAIL_ROWS), :],
                        o_hbm.at[pl.ds(chunk * N_NODES + t0, TAIL_ROWS), :])

    return k(xflat, src_off, dst2)


def _tc_mlp(hpre, W1, b1, W2, b2):
    nchin = hpre.shape[0]
    h_mid = W1.shape[1]
    dout = W2.shape[1]
    nchout = dout // 128
    R = 2000
    G = N_NODES // R

    def body(h_ref, w1_ref, b1_ref, w2_ref, b2_ref, o_ref):
        s = jnp.dot(h_ref[0], w1_ref[0:128, :],
                    preferred_element_type=jnp.float32)
        for c in range(1, nchin):
            s = s + jnp.dot(h_ref[c], w1_ref[128 * c:128 * (c + 1), :],
                            preferred_element_type=jnp.float32)
        m = jnp.maximum(s + b1_ref[...], 0.0)
        o = jnp.dot(m, w2_ref[...], preferred_element_type=jnp.float32)
        o = o + b2_ref[...]
        for c in range(nchout):
            o_ref[c, :, :] = o[:, 128 * c:128 * (c + 1)]

    return pl.pallas_call(
        body,
        grid=(G,),
        in_specs=[
            pl.BlockSpec((nchin, R, 128), lambda i: (0, i, 0)),
            pl.BlockSpec((nchin * 128, h_mid), lambda i: (0, 0)),
            pl.BlockSpec((1, h_mid), lambda i: (0, 0)),
            pl.BlockSpec((h_mid, dout), lambda i: (0, 0)),
            pl.BlockSpec((1, dout), lambda i: (0, 0)),
        ],
        out_specs=pl.BlockSpec((nchout, R, 128), lambda i: (0, i, 0)),
        out_shape=jax.ShapeDtypeStruct((nchout, N_NODES, 128), jnp.float32),
    )(hpre, W1, b1.reshape(1, h_mid), W2, b2.reshape(1, dout))


def _tc_final(hpre, W3a, b3a, W3b, b3b):
    nchin = hpre.shape[0]
    h_mid = W3a.shape[1]
    dout = W3b.shape[1]
    R = 2000
    G = N_NODES // R

    def body(h_ref, w3a_ref, b3a_ref, w3b_ref, b3b_ref, o_ref, acc_ref):
        i = pl.program_id(0)
        s = jnp.dot(h_ref[0], w3a_ref[0:128, :],
                    preferred_element_type=jnp.float32)
        for c in range(1, nchin):
            s = s + jnp.dot(h_ref[c], w3a_ref[128 * c:128 * (c + 1), :],
                            preferred_element_type=jnp.float32)
        m = jnp.maximum(s + b3a_ref[...], 0.0)
        part = jnp.sum(m, axis=0, keepdims=True)

        @pl.when(i == 0)
        def _():
            acc_ref[...] = part

        @pl.when(i > 0)
        def _():
            acc_ref[...] = acc_ref[...] + part

        @pl.when(i == G - 1)
        def _():
            o_ref[...] = (jnp.dot(acc_ref[...], w3b_ref[...],
                                  preferred_element_type=jnp.float32)
                          + float(N_NODES) * b3b_ref[...])

    return pl.pallas_call(
        body,
        grid=(G,),
        in_specs=[
            pl.BlockSpec((nchin, R, 128), lambda i: (0, i, 0)),
            pl.BlockSpec((nchin * 128, h_mid), lambda i: (0, 0)),
            pl.BlockSpec((1, h_mid), lambda i: (0, 0)),
            pl.BlockSpec((h_mid, dout), lambda i: (0, 0)),
            pl.BlockSpec((1, dout), lambda i: (0, 0)),
        ],
        out_specs=pl.BlockSpec((1, dout), lambda i: (0, 0)),
        out_shape=jax.ShapeDtypeStruct((1, dout), jnp.float32),
        scratch_shapes=[pltpu.VMEM((1, h_mid), jnp.float32)],
    )(hpre, W3a, b3a.reshape(1, h_mid), W3b, b3b.reshape(1, dout))


def kernel(x, edge_index, edge_attr, W1a, b1a, W1b, b1b, W2a, b2a, W2b, b2b,
           W3a, b3a, W3b, b3b):
    del edge_attr
    src = edge_index[0].astype(jnp.int32)
    dst = edge_index[1].astype(jnp.int32)
    pad = EDGE_PAD - N_EDGES
    srcp = jnp.concatenate([src, jnp.zeros((pad,), jnp.int32)])
    dstp = jnp.concatenate([dst, jnp.full((pad,), N_NODES, jnp.int32)])
    dst2 = dstp.reshape(EDGE_ROWS, 128)
    src2 = srcp.reshape(EDGE_ROWS, 128)
    off2 = (jnp.arange(2, dtype=jnp.int32) * N_NODES)[:, None, None]
    off4 = (jnp.arange(4, dtype=jnp.int32) * N_NODES)[:, None, None]
    src_off2 = src2[None] + off2
    src_off4 = src2[None] + off4

    x2 = x.reshape(N_NODES, 2, 128).transpose(1, 0, 2)

    hpre1 = _sc_aggregate(x2.reshape(2 * N_NODES, 128), src_off2, dst2, 2)
    h1 = _tc_mlp(hpre1.reshape(2, N_NODES, 128), W1a, b1a, W1b, b1b)

    hpre2 = _sc_aggregate(h1.reshape(4 * N_NODES, 128), src_off4, dst2, 4)
    h2 = _tc_mlp(hpre2.reshape(4, N_NODES, 128), W2a, b2a, W2b, b2b)

    hpre3 = _sc_aggregate(h2.reshape(4 * N_NODES, 128), src_off4, dst2, 4)
    out = _tc_final(hpre3.reshape(4, N_NODES, 128), W3a, b3a, W3b, b3b)
    return out
